# Initial kernel scaffold; baseline (speedup 1.0000x reference)
import jax
import jax.numpy as jnp
from jax.experimental import pallas as pl


def kernel(tokens, lengths, queries, ln_gamma, ln_beta, mlp1_W1, mlp1_b1, mlp1_W2, mlp1_b2, mlp2_W1, mlp2_b1, mlp2_W2, mlp2_b2):
    raise NotImplementedError("write your pallas kernel here")


# speedup vs baseline: 2.6333x; 2.6333x over previous
import functools

import jax
import jax.numpy as jnp
from jax import lax
from jax.experimental import pallas as pl
from jax.experimental.pallas import tpu as pltpu

B, S, D = 16, 4096, 1024
M = 2
CHUNK = 256
NEG = -1e30
POS = 1e30


def _pool_body(lengths_ref, G_ref, c0_ref, bq_ref, tokens_hbm,
               trad_ref, learn_ref,
               buf, clf_buf, tail_buf, sum_acc, max_acc, min_acc, att_acc,
               m_acc, l_acc, sems):
    b = pl.program_id(0)
    L = lengths_ref[b]
    n = lax.min(lax.div(L + CHUNK, CHUNK), S // CHUNK)
    has_tail = L >= S

    pltpu.make_async_copy(tokens_hbm.at[b, pl.ds(0, CHUNK), :], buf.at[0],
                          sems.at[0]).start()

    @pl.when(has_tail)
    def _tail_start():
        pltpu.make_async_copy(tokens_hbm.at[b, pl.ds(S, 1), :], tail_buf,
                              sems.at[2]).start()

    sum_acc[...] = jnp.zeros_like(sum_acc)
    max_acc[...] = jnp.full_like(max_acc, NEG)
    min_acc[...] = jnp.full_like(min_acc, POS)
    att_acc[...] = jnp.zeros_like(att_acc)
    m_acc[...] = jnp.full_like(m_acc, NEG)
    l_acc[...] = jnp.zeros_like(l_acc)

    def chunk_step(k, _):
        slot = lax.rem(k, 2)

        @pl.when(k + 1 < n)
        def _prefetch():
            nslot = lax.rem(k + 1, 2)
            pltpu.make_async_copy(
                tokens_hbm.at[b, pl.ds((k + 1) * CHUNK, CHUNK), :],
                buf.at[nslot], sems.at[nslot]).start()

        pltpu.make_async_copy(
            tokens_hbm.at[b, pl.ds(k * CHUNK, CHUNK), :],
            buf.at[slot], sems.at[slot]).wait()

        x = buf[slot]
        g = k * CHUNK + lax.broadcasted_iota(jnp.int32, (CHUNK, 1), 0)
        rmask = (g >= 1) & (g <= L)
        maskf = rmask.astype(jnp.float32)

        @pl.when(k == 0)
        def _grab_clf():
            clf_buf[...] = x[0:1, :]

        sum_acc[...] += lax.dot_general(
            maskf, x, (((0,), (0,)), ((), ())),
            preferred_element_type=jnp.float32)

        full = (k >= 1) & (L >= (k + 1) * CHUNK - 1)

        @pl.when(full)
        def _full():
            max_acc[...] = jnp.maximum(max_acc[...],
                                       jnp.max(x, axis=0, keepdims=True))
            min_acc[...] = jnp.minimum(min_acc[...],
                                       jnp.min(x, axis=0, keepdims=True))

        @pl.when(jnp.logical_not(full))
        def _partial():
            max_acc[...] = jnp.maximum(
                max_acc[...],
                jnp.max(jnp.where(rmask, x, NEG), axis=0, keepdims=True))
            min_acc[...] = jnp.minimum(
                min_acc[...],
                jnp.min(jnp.where(rmask, x, POS), axis=0, keepdims=True))

        xg = lax.dot_general(x, G_ref[...], (((1,), (0,)), ((), ())),
                             preferred_element_type=jnp.float32)
        sq = lax.dot_general(x * x, G_ref[...], (((1,), (0,)), ((), ())),
                             preferred_element_type=jnp.float32)
        mu = xg[:, M:M + 1]
        var = sq[:, M:M + 1] - mu * mu
        rsq = lax.rsqrt(var + 1e-5) * (D ** -0.5)
        st = rsq * (xg[:, 0:M] - mu * c0_ref[...]) + bq_ref[...]
        st = jnp.where(rmask, st, NEG)
        cmax = jnp.max(st, axis=0, keepdims=True)
        new_m = jnp.maximum(m_acc[...], cmax)
        alpha = jnp.exp(m_acc[...] - new_m)
        p = jnp.exp(st - new_m)
        l_acc[...] = l_acc[...] * alpha + jnp.sum(p, axis=0, keepdims=True)
        att_acc[...] = att_acc[...] * alpha.reshape(M, 1) + lax.dot_general(
            p, x, (((0,), (0,)), ((), ())),
            preferred_element_type=jnp.float32)
        m_acc[...] = new_m

    lax.fori_loop(0, n, chunk_step, None)

    @pl.when(has_tail)
    def _tail():
        pltpu.make_async_copy(tokens_hbm.at[b, pl.ds(S, 1), :], tail_buf,
                              sems.at[2]).wait()
        xt = tail_buf[...]
        sum_acc[...] += xt
        max_acc[...] = jnp.maximum(max_acc[...], xt)
        min_acc[...] = jnp.minimum(min_acc[...], xt)
        xg = lax.dot_general(xt, G_ref[...], (((1,), (0,)), ((), ())),
                             preferred_element_type=jnp.float32)
        sq = lax.dot_general(xt * xt, G_ref[...], (((1,), (0,)), ((), ())),
                             preferred_element_type=jnp.float32)
        mu = xg[:, M:M + 1]
        var = sq[:, M:M + 1] - mu * mu
        rsq = lax.rsqrt(var + 1e-5) * (D ** -0.5)
        st = rsq * (xg[:, 0:M] - mu * c0_ref[...]) + bq_ref[...]
        new_m = jnp.maximum(m_acc[...], st)
        alpha = jnp.exp(m_acc[...] - new_m)
        p = jnp.exp(st - new_m)
        l_acc[...] = l_acc[...] * alpha + p
        att_acc[...] = (att_acc[...] * alpha.reshape(M, 1)
                        + p.reshape(M, 1) * xt)
        m_acc[...] = new_m
    trad_ref[0, 0:1, 0:D] = sum_acc[...] / L.astype(jnp.float32)
    trad_ref[0, 0:1, D:2 * D] = max_acc[...]
    trad_ref[0, 0:1, 2 * D:3 * D] = min_acc[...]
    pmp = att_acc[...] / l_acc[...].reshape(M, 1)
    learn_ref[0, 0:1, 0:D] = pmp[0:1, :]
    learn_ref[0, 0:1, D:2 * D] = pmp[1:2, :]
    learn_ref[0, 0:1, 2 * D:3 * D] = clf_buf[...]


def _gelu_exact(x):
    return x * 0.5 * (1.0 + lax.erf(x * (2.0 ** -0.5)))


def _mlp_body(x1_ref, x2_ref, w11_ref, b11_ref, w12_ref, b12_ref,
              w21_ref, b21_ref, w22_ref, b22_ref, out_ref):
    h1 = _gelu_exact(
        jnp.dot(x1_ref[...], w11_ref[...],
                preferred_element_type=jnp.float32) + b11_ref[...])
    out_ref[:, 0:D] = jnp.dot(
        h1, w12_ref[...], preferred_element_type=jnp.float32) + b12_ref[...]
    h2 = _gelu_exact(
        jnp.dot(x2_ref[...], w21_ref[...],
                preferred_element_type=jnp.float32) + b21_ref[...])
    out_ref[:, D:2 * D] = jnp.dot(
        h2, w22_ref[...], preferred_element_type=jnp.float32) + b22_ref[...]


@jax.jit
def kernel(tokens, lengths, queries, ln_gamma, ln_beta,
           mlp1_W1, mlp1_b1, mlp1_W2, mlp1_b2,
           mlp2_W1, mlp2_b1, mlp2_W2, mlp2_b2):
    lengths = lengths.astype(jnp.int32)
    qg = (queries * ln_gamma[None, :]).T
    G = jnp.concatenate(
        [qg, jnp.full((D, 1), 1.0 / D, jnp.float32)], axis=1)
    c0 = jnp.sum(qg, axis=0).reshape(1, M)
    bq = (queries @ ln_beta).reshape(1, M) * (D ** -0.5)

    trad, learn = pl.pallas_call(
        _pool_body,
        grid=(B,),
        in_specs=[
            pl.BlockSpec(memory_space=pltpu.SMEM),
            pl.BlockSpec(memory_space=pltpu.VMEM),
            pl.BlockSpec(memory_space=pltpu.VMEM),
            pl.BlockSpec(memory_space=pltpu.VMEM),
            pl.BlockSpec(memory_space=pltpu.MemorySpace.HBM),
        ],
        out_specs=[
            pl.BlockSpec((1, 1, 3 * D), lambda b: (b, 0, 0)),
            pl.BlockSpec((1, 1, 3 * D), lambda b: (b, 0, 0)),
        ],
        out_shape=[
            jax.ShapeDtypeStruct((B, 1, 3 * D), jnp.float32),
            jax.ShapeDtypeStruct((B, 1, 3 * D), jnp.float32),
        ],
        scratch_shapes=[
            pltpu.VMEM((2, CHUNK, D), jnp.float32),
            pltpu.VMEM((1, D), jnp.float32),
            pltpu.VMEM((1, D), jnp.float32),
            pltpu.VMEM((1, D), jnp.float32),
            pltpu.VMEM((1, D), jnp.float32),
            pltpu.VMEM((1, D), jnp.float32),
            pltpu.VMEM((M, D), jnp.float32),
            pltpu.VMEM((1, M), jnp.float32),
            pltpu.VMEM((1, M), jnp.float32),
            pltpu.SemaphoreType.DMA((3,)),
        ],
        compiler_params=pltpu.CompilerParams(
            dimension_semantics=("arbitrary",)),
    )(lengths, G, c0, bq, tokens)

    out = pl.pallas_call(
        _mlp_body,
        out_shape=jax.ShapeDtypeStruct((B, 2 * D), jnp.float32),
    )(trad.reshape(B, 3 * D), learn.reshape(B, 3 * D),
      mlp1_W1, mlp1_b1.reshape(1, D), mlp1_W2, mlp1_b2.reshape(1, D),
      mlp2_W1, mlp2_b1.reshape(1, D), mlp2_W2, mlp2_b2.reshape(1, D))
    return out


# speedup vs baseline: 2.6956x; 1.0237x over previous
import jax
import jax.numpy as jnp
from jax import lax
from jax.experimental import pallas as pl
from jax.experimental.pallas import tpu as pltpu

B, S, D = 16, 4096, 1024
M = 2
CHUNK = 256
NBLK = S // CHUNK + 1
NEG = -1e30
POS = 1e30


def _pool_body(lens_ref, G_ref, c0_ref, bq_ref, tok_ref,
               trad_ref, learn_ref,
               clf_buf, sum_acc, max_acc, min_acc, att_acc, m_acc, l_acc):
    b = pl.program_id(0)
    j = pl.program_id(1)
    L = lens_ref[b]
    jlast = lax.div(L + CHUNK, CHUNK) - 1

    @pl.when(j == 0)
    def _init():
        sum_acc[...] = jnp.zeros_like(sum_acc)
        max_acc[...] = jnp.full_like(max_acc, NEG)
        min_acc[...] = jnp.full_like(min_acc, POS)
        att_acc[...] = jnp.zeros_like(att_acc)
        m_acc[...] = jnp.full_like(m_acc, NEG)
        l_acc[...] = jnp.zeros_like(l_acc)
        clf_buf[...] = tok_ref[0, 0:1, :]

    @pl.when(j <= jlast)
    def _accumulate():
        x = tok_ref[0]
        g = j * CHUNK + lax.broadcasted_iota(jnp.int32, (CHUNK, 1), 0)
        rmask = (g >= 1) & (g <= L)
        ones = jnp.ones((1, CHUNK), jnp.float32)
        full = (j >= 1) & (L >= (j + 1) * CHUNK - 1)

        def _attention(x, xz, rmask):
            xg = lax.dot_general(x, G_ref[...], (((1,), (0,)), ((), ())),
                                 preferred_element_type=jnp.float32)
            sq = lax.dot_general(x * x, G_ref[...], (((1,), (0,)), ((), ())),
                                 preferred_element_type=jnp.float32)
            mu = xg[:, M:M + 1]
            var = sq[:, M:M + 1] - mu * mu
            rsq = lax.rsqrt(var + 1e-5) * (D ** -0.5)
            st = rsq * (xg[:, 0:M] - mu * c0_ref[...]) + bq_ref[...]
            st = jnp.where(rmask, st, NEG)
            cmax = jnp.max(st, axis=0, keepdims=True)
            new_m = jnp.maximum(m_acc[...], cmax)
            alpha = jnp.exp(m_acc[...] - new_m)
            p = jnp.exp(st - new_m)
            l_acc[...] = (l_acc[...] * alpha
                          + jnp.sum(p, axis=0, keepdims=True))
            att_acc[...] = (att_acc[...] * alpha.reshape(M, 1)
                            + lax.dot_general(
                                p, xz, (((0,), (0,)), ((), ())),
                                preferred_element_type=jnp.float32))
            m_acc[...] = new_m

        @pl.when(full)
        def _full():
            sum_acc[...] += lax.dot_general(
                ones, x, (((1,), (0,)), ((), ())),
                preferred_element_type=jnp.float32)
            max_acc[...] = jnp.maximum(max_acc[...],
                                       jnp.max(x, axis=0, keepdims=True))
            min_acc[...] = jnp.minimum(min_acc[...],
                                       jnp.min(x, axis=0, keepdims=True))
            _attention(x, x, rmask)

        @pl.when(jnp.logical_not(full))
        def _partial():
            xz = jnp.where(rmask, x, 0.0)
            sum_acc[...] += lax.dot_general(
                ones, xz, (((1,), (0,)), ((), ())),
                preferred_element_type=jnp.float32)
            max_acc[...] = jnp.maximum(
                max_acc[...],
                jnp.max(jnp.where(rmask, x, NEG), axis=0, keepdims=True))
            min_acc[...] = jnp.minimum(
                min_acc[...],
                jnp.min(jnp.where(rmask, x, POS), axis=0, keepdims=True))
            _attention(x, xz, rmask)

    @pl.when(j == NBLK - 1)
    def _finalize():
        trad_ref[0, 0:1, 0:D] = sum_acc[...] / L.astype(jnp.float32)
        trad_ref[0, 0:1, D:2 * D] = max_acc[...]
        trad_ref[0, 0:1, 2 * D:3 * D] = min_acc[...]
        pmp = att_acc[...] / l_acc[...].reshape(M, 1)
        learn_ref[0, 0:1, 0:D] = pmp[0:1, :]
        learn_ref[0, 0:1, D:2 * D] = pmp[1:2, :]
        learn_ref[0, 0:1, 2 * D:3 * D] = clf_buf[...]


def _gelu_exact(x):
    return x * 0.5 * (1.0 + lax.erf(x * (2.0 ** -0.5)))


def _mlp_body(x1_ref, x2_ref, w11_ref, b11_ref, w12_ref, b12_ref,
              w21_ref, b21_ref, w22_ref, b22_ref, out_ref):
    h1 = _gelu_exact(
        jnp.dot(x1_ref[...], w11_ref[...],
                preferred_element_type=jnp.float32) + b11_ref[...])
    out_ref[:, 0:D] = jnp.dot(
        h1, w12_ref[...], preferred_element_type=jnp.float32) + b12_ref[...]
    h2 = _gelu_exact(
        jnp.dot(x2_ref[...], w21_ref[...],
                preferred_element_type=jnp.float32) + b21_ref[...])
    out_ref[:, D:2 * D] = jnp.dot(
        h2, w22_ref[...], preferred_element_type=jnp.float32) + b22_ref[...]


def _tok_index(b, j, lens):
    jl = lax.div(lens[b] + CHUNK, CHUNK) - 1
    return (b, jnp.minimum(j, jl), 0)


@jax.jit
def kernel(tokens, lengths, queries, ln_gamma, ln_beta,
           mlp1_W1, mlp1_b1, mlp1_W2, mlp1_b2,
           mlp2_W1, mlp2_b1, mlp2_W2, mlp2_b2):
    lengths = lengths.astype(jnp.int32)
    qg = (queries * ln_gamma[None, :]).T
    G = jnp.concatenate(
        [qg, jnp.full((D, 1), 1.0 / D, jnp.float32)], axis=1)
    c0 = jnp.sum(qg, axis=0).reshape(1, M)
    bq = (queries @ ln_beta).reshape(1, M) * (D ** -0.5)

    grid_spec = pltpu.PrefetchScalarGridSpec(
        num_scalar_prefetch=1,
        grid=(B, NBLK),
        in_specs=[
            pl.BlockSpec(memory_space=pltpu.VMEM),
            pl.BlockSpec(memory_space=pltpu.VMEM),
            pl.BlockSpec(memory_space=pltpu.VMEM),
            pl.BlockSpec((1, CHUNK, D), _tok_index),
        ],
        out_specs=[
            pl.BlockSpec((1, 1, 3 * D), lambda b, j, lens: (b, 0, 0)),
            pl.BlockSpec((1, 1, 3 * D), lambda b, j, lens: (b, 0, 0)),
        ],
        scratch_shapes=[
            pltpu.VMEM((1, D), jnp.float32),
            pltpu.VMEM((1, D), jnp.float32),
            pltpu.VMEM((1, D), jnp.float32),
            pltpu.VMEM((1, D), jnp.float32),
            pltpu.VMEM((M, D), jnp.float32),
            pltpu.VMEM((1, M), jnp.float32),
            pltpu.VMEM((1, M), jnp.float32),
        ],
    )
    trad, learn = pl.pallas_call(
        _pool_body,
        grid_spec=grid_spec,
        out_shape=[
            jax.ShapeDtypeStruct((B, 1, 3 * D), jnp.float32),
            jax.ShapeDtypeStruct((B, 1, 3 * D), jnp.float32),
        ],
        compiler_params=pltpu.CompilerParams(
            dimension_semantics=("arbitrary", "arbitrary")),
    )(lengths, G, c0, bq, tokens)

    out = pl.pallas_call(
        _mlp_body,
        out_shape=jax.ShapeDtypeStruct((B, 2 * D), jnp.float32),
    )(trad.reshape(B, 3 * D), learn.reshape(B, 3 * D),
      mlp1_W1, mlp1_b1.reshape(1, D), mlp1_W2, mlp1_b2.reshape(1, D),
      mlp2_W1, mlp2_b1.reshape(1, D), mlp2_W2, mlp2_b2.reshape(1, D))
    return out


# speedup vs baseline: 2.9315x; 1.0875x over previous
import jax
import jax.numpy as jnp
from jax import lax
from jax.experimental import pallas as pl
from jax.experimental.pallas import tpu as pltpu

B, S, D = 16, 4096, 1024
M = 2
CHUNK = 512
NBLK = S // CHUNK + 1
NEG = -1e30
POS = 1e30


def _pool_body(lens_ref, G_ref, c0_ref, bq_ref, tok_ref,
               trad_ref, learn_ref,
               clf_buf, sum_acc, max_acc, min_acc, att_acc, m_acc, l_acc):
    b = pl.program_id(0)
    j = pl.program_id(1)
    L = lens_ref[b]
    jlast = lax.div(L + CHUNK, CHUNK) - 1

    @pl.when(j == 0)
    def _init():
        sum_acc[...] = jnp.zeros_like(sum_acc)
        max_acc[...] = jnp.full_like(max_acc, NEG)
        min_acc[...] = jnp.full_like(min_acc, POS)
        att_acc[...] = jnp.zeros_like(att_acc)
        m_acc[...] = jnp.full_like(m_acc, NEG)
        l_acc[...] = jnp.zeros_like(l_acc)
        clf_buf[...] = tok_ref[0, 0:1, :]

    @pl.when(j <= jlast)
    def _accumulate():
        x = tok_ref[0]
        g = j * CHUNK + lax.broadcasted_iota(jnp.int32, (CHUNK, 1), 0)
        rmask = (g >= 1) & (g <= L)
        ones = jnp.ones((1, CHUNK), jnp.float32)
        full = (j >= 1) & (L >= (j + 1) * CHUNK - 1)

        def _attention(x, xz, rmask):
            xg = lax.dot_general(x, G_ref[...], (((1,), (0,)), ((), ())),
                                 preferred_element_type=jnp.float32)
            sq = lax.dot_general(x * x, G_ref[...], (((1,), (0,)), ((), ())),
                                 preferred_element_type=jnp.float32)
            mu = xg[:, M:M + 1]
            var = sq[:, M:M + 1] - mu * mu
            rsq = lax.rsqrt(var + 1e-5) * (D ** -0.5)
            st = rsq * (xg[:, 0:M] - mu * c0_ref[...]) + bq_ref[...]
            st = jnp.where(rmask, st, NEG)
            cmax = jnp.max(st, axis=0, keepdims=True)
            new_m = jnp.maximum(m_acc[...], cmax)
            alpha = jnp.exp(m_acc[...] - new_m)
            p = jnp.exp(st - new_m)
            l_acc[...] = (l_acc[...] * alpha
                          + jnp.sum(p, axis=0, keepdims=True))
            att_acc[...] = (att_acc[...] * alpha.reshape(M, 1)
                            + lax.dot_general(
                                p, xz, (((0,), (0,)), ((), ())),
                                preferred_element_type=jnp.float32))
            m_acc[...] = new_m

        @pl.when(full)
        def _full():
            sum_acc[...] += lax.dot_general(
                ones, x, (((1,), (0,)), ((), ())),
                preferred_element_type=jnp.float32)
            max_acc[...] = jnp.maximum(max_acc[...],
                                       jnp.max(x, axis=0, keepdims=True))
            min_acc[...] = jnp.minimum(min_acc[...],
                                       jnp.min(x, axis=0, keepdims=True))
            _attention(x, x, rmask)

        @pl.when(jnp.logical_not(full))
        def _partial():
            xz = jnp.where(rmask, x, 0.0)
            sum_acc[...] += lax.dot_general(
                ones, xz, (((1,), (0,)), ((), ())),
                preferred_element_type=jnp.float32)
            max_acc[...] = jnp.maximum(
                max_acc[...],
                jnp.max(jnp.where(rmask, x, NEG), axis=0, keepdims=True))
            min_acc[...] = jnp.minimum(
                min_acc[...],
                jnp.min(jnp.where(rmask, x, POS), axis=0, keepdims=True))
            _attention(x, xz, rmask)

    @pl.when(j == NBLK - 1)
    def _finalize():
        trad_ref[0, 0:1, 0:D] = sum_acc[...] / L.astype(jnp.float32)
        trad_ref[0, 0:1, D:2 * D] = max_acc[...]
        trad_ref[0, 0:1, 2 * D:3 * D] = min_acc[...]
        pmp = att_acc[...] / l_acc[...].reshape(M, 1)
        learn_ref[0, 0:1, 0:D] = pmp[0:1, :]
        learn_ref[0, 0:1, D:2 * D] = pmp[1:2, :]
        learn_ref[0, 0:1, 2 * D:3 * D] = clf_buf[...]


def _gelu_exact(x):
    return x * 0.5 * (1.0 + lax.erf(x * (2.0 ** -0.5)))


def _mlp_body(x1_ref, x2_ref, w11_ref, b11_ref, w12_ref, b12_ref,
              w21_ref, b21_ref, w22_ref, b22_ref, out_ref):
    h1 = _gelu_exact(
        jnp.dot(x1_ref[...], w11_ref[...],
                preferred_element_type=jnp.float32) + b11_ref[...])
    out_ref[:, 0:D] = jnp.dot(
        h1, w12_ref[...], preferred_element_type=jnp.float32) + b12_ref[...]
    h2 = _gelu_exact(
        jnp.dot(x2_ref[...], w21_ref[...],
                preferred_element_type=jnp.float32) + b21_ref[...])
    out_ref[:, D:2 * D] = jnp.dot(
        h2, w22_ref[...], preferred_element_type=jnp.float32) + b22_ref[...]


def _tok_index(b, j, lens):
    jl = lax.div(lens[b] + CHUNK, CHUNK) - 1
    return (b, jnp.minimum(j, jl), 0)


@jax.jit
def kernel(tokens, lengths, queries, ln_gamma, ln_beta,
           mlp1_W1, mlp1_b1, mlp1_W2, mlp1_b2,
           mlp2_W1, mlp2_b1, mlp2_W2, mlp2_b2):
    lengths = lengths.astype(jnp.int32)
    qg = (queries * ln_gamma[None, :]).T
    G = jnp.concatenate(
        [qg, jnp.full((D, 1), 1.0 / D, jnp.float32)], axis=1)
    c0 = jnp.sum(qg, axis=0).reshape(1, M)
    bq = (queries @ ln_beta).reshape(1, M) * (D ** -0.5)

    grid_spec = pltpu.PrefetchScalarGridSpec(
        num_scalar_prefetch=1,
        grid=(B, NBLK),
        in_specs=[
            pl.BlockSpec(memory_space=pltpu.VMEM),
            pl.BlockSpec(memory_space=pltpu.VMEM),
            pl.BlockSpec(memory_space=pltpu.VMEM),
            pl.BlockSpec((1, CHUNK, D), _tok_index),
        ],
        out_specs=[
            pl.BlockSpec((1, 1, 3 * D), lambda b, j, lens: (b, 0, 0)),
            pl.BlockSpec((1, 1, 3 * D), lambda b, j, lens: (b, 0, 0)),
        ],
        scratch_shapes=[
            pltpu.VMEM((1, D), jnp.float32),
            pltpu.VMEM((1, D), jnp.float32),
            pltpu.VMEM((1, D), jnp.float32),
            pltpu.VMEM((1, D), jnp.float32),
            pltpu.VMEM((M, D), jnp.float32),
            pltpu.VMEM((1, M), jnp.float32),
            pltpu.VMEM((1, M), jnp.float32),
        ],
    )
    trad, learn = pl.pallas_call(
        _pool_body,
        grid_spec=grid_spec,
        out_shape=[
            jax.ShapeDtypeStruct((B, 1, 3 * D), jnp.float32),
            jax.ShapeDtypeStruct((B, 1, 3 * D), jnp.float32),
        ],
        compiler_params=pltpu.CompilerParams(
            dimension_semantics=("arbitrary", "arbitrary")),
    )(lengths, G, c0, bq, tokens)

    out = pl.pallas_call(
        _mlp_body,
        out_shape=jax.ShapeDtypeStruct((B, 2 * D), jnp.float32),
    )(trad.reshape(B, 3 * D), learn.reshape(B, 3 * D),
      mlp1_W1, mlp1_b1.reshape(1, D), mlp1_W2, mlp1_b2.reshape(1, D),
      mlp2_W1, mlp2_b1.reshape(1, D), mlp2_W2, mlp2_b2.reshape(1, D))
    return out
